# baseline (speedup 1.0000x reference)
import functools

import jax
import jax.numpy as jnp
import numpy as np
from jax import lax
from jax.experimental import pallas as pl
from jax.experimental.pallas import tpu as pltpu
from jax.experimental.pallas import tpu_sc as plsc

B = 16384
C = 1000
SMOOTHING = 0.1
CONF = 1.0 - SMOOTHING
OFF = SMOOTHING / (C - 1)
KLSE = CONF - OFF + OFF * C
ENT = CONF * float(np.log(CONF)) + (C - 1) * OFF * float(np.log(OFF))

B_TC = 10240
B_SC = B - B_TC
ROWS_PER_BLOCK = 2048
NCHUNK = (C + 15) // 16


def _dense_body(x_ref, lab_ref, acc_ref):
    i = pl.program_id(0)
    x = x_ref[...]
    lab = lab_ref[0, 0, :]
    m = jnp.max(x, axis=1, keepdims=True)
    lse = m + jnp.log(jnp.sum(jnp.exp(x - m), axis=1, keepdims=True))
    col = lax.broadcasted_iota(jnp.int32, (ROWS_PER_BLOCK, C), 1)
    g = jnp.sum(jnp.where(col == lab[:, None], x, 0.0))
    partial = jnp.reshape(
        OFF * jnp.sum(x) - KLSE * jnp.sum(lse) + (CONF - OFF) * g, (1, 1)
    )

    @pl.when(i == 0)
    def _():
        acc_ref[...] = jnp.zeros((1, 1), jnp.float32)

    acc_ref[...] += partial


_dense_call = pl.pallas_call(
    _dense_body,
    grid=(B_TC // ROWS_PER_BLOCK,),
    in_specs=[
        pl.BlockSpec((ROWS_PER_BLOCK, C), lambda i: (i, 0)),
        pl.BlockSpec((1, 1, ROWS_PER_BLOCK), lambda i: (i, 0, 0)),
    ],
    out_specs=pl.BlockSpec((1, 1), lambda i: (0, 0)),
    out_shape=jax.ShapeDtypeStruct((1, 1), jnp.float32),
)


_INFO = plsc.get_sparse_core_info()
_NC = _INFO.num_cores
_NS = _INFO.num_subcores
_NW = _NC * _NS
_TR = B_SC // _NW
_SLAB = 8
_NSLAB = _TR // _SLAB
_NFULL = C // 16
_TAILOFF = C - 16

_sc_mesh = plsc.VectorSubcoreMesh(core_axis_name="c", subcore_axis_name="s")


@functools.partial(
    pl.kernel,
    out_type=jax.ShapeDtypeStruct((4, B_SC, 16), jnp.float32),
    mesh=_sc_mesh,
    scratch_types=[
        pltpu.VMEM((_TR,), jnp.int32),
        pltpu.VMEM((_TR, 16), jnp.int32),
        pltpu.VMEM((_SLAB, C), jnp.float32),
        pltpu.VMEM((4, _SLAB, 16), jnp.float32),
    ],
)
def _sc_dense(x_hbm, labels_hbm, out_hbm, lab_v, splat_v, xbuf, st_v):
    cc = lax.axis_index("c")
    ss = lax.axis_index("s")
    wid = ss * _NC + cc
    r0 = B_TC + wid * _TR
    pltpu.sync_copy(labels_hbm.at[pl.ds(r0, _TR)], lab_v)
    for j in range(_TR // 16):
        chunk = lab_v[pl.ds(j * 16, 16)]
        for r in range(16):
            splat_v[j * 16 + r, :] = jnp.broadcast_to(chunk[r], (16,))
    iota = lax.iota(jnp.int32, 16)
    fresh = iota >= (16 - (C - _NFULL * 16))

    def slab_body(sl, _):
        row = r0 + sl * _SLAB
        pltpu.sync_copy(x_hbm.at[pl.ds(row, _SLAB), :], xbuf)
        for r in range(_SLAB):
            m = jnp.full((16,), -jnp.inf, jnp.float32)
            for cn in range(_NFULL):
                m = jnp.maximum(m, xbuf[r, pl.ds(cn * 16, 16)])
            xt = xbuf[r, pl.ds(_TAILOFF, 16)]
            m = jnp.maximum(m, jnp.where(fresh, xt, -jnp.inf))
            s_l = jnp.zeros((16,), jnp.float32)
            rs_l = jnp.zeros((16,), jnp.float32)
            for cn in range(_NFULL):
                x = xbuf[r, pl.ds(cn * 16, 16)]
                s_l = s_l + jnp.exp(x - m)
                rs_l = rs_l + x
            s_l = s_l + jnp.exp(jnp.where(fresh, xt, -jnp.inf) - m)
            rs_l = rs_l + jnp.where(fresh, xt, 0.0)
            lab_s = splat_v[sl * _SLAB + r, :]
            cstartv = lax.shift_left(lax.shift_right_logical(lab_s, 4), 4)
            xg = xbuf[r, pl.ds(pl.multiple_of(cstartv[0], 16), 16)]
            g_l = jnp.where(cstartv + iota == lab_s, xg, 0.0)
            st_v[0, r, :] = m
            st_v[1, r, :] = s_l
            st_v[2, r, :] = rs_l
            st_v[3, r, :] = g_l
        pltpu.sync_copy(
            st_v, out_hbm.at[:, pl.ds(wid * _TR + sl * _SLAB, _SLAB), :]
        )
        return 0

    lax.fori_loop(0, _NSLAB, slab_body, 0)


def _finish_body(st_ref, acc_ref):
    m_l = st_ref[0]
    s_l = st_ref[1]
    rs_l = st_ref[2]
    g_l = st_ref[3]
    mm = jnp.max(m_l, axis=1, keepdims=True)
    s_tot = jnp.sum(s_l * jnp.exp(m_l - mm), axis=1, keepdims=True)
    lse = mm + jnp.log(s_tot)
    acc_ref[...] = jnp.reshape(
        OFF * jnp.sum(rs_l) - KLSE * jnp.sum(lse) + (CONF - OFF) * jnp.sum(g_l),
        (1, 1),
    )


_finish_call = pl.pallas_call(
    _finish_body,
    out_shape=jax.ShapeDtypeStruct((1, 1), jnp.float32),
)


@jax.jit
def kernel(outputs, labels):
    lab32 = labels.astype(jnp.int32)
    stats = _sc_dense(outputs, lab32)
    lab3 = jnp.reshape(lab32, (B // ROWS_PER_BLOCK, 1, ROWS_PER_BLOCK))
    acc_tc = _dense_call(outputs, lab3)[0, 0]
    acc_sc = _finish_call(stats)[0, 0]
    return ENT - (acc_tc + acc_sc) / B


# speedup vs baseline: 1.0017x; 1.0017x over previous
import functools

import jax
import jax.numpy as jnp
import numpy as np
from jax import lax
from jax.experimental import pallas as pl
from jax.experimental.pallas import tpu as pltpu
from jax.experimental.pallas import tpu_sc as plsc

B = 16384
C = 1000
SMOOTHING = 0.1
CONF = 1.0 - SMOOTHING
OFF = SMOOTHING / (C - 1)
KLSE = CONF - OFF + OFF * C
ENT = CONF * float(np.log(CONF)) + (C - 1) * OFF * float(np.log(OFF))

B_TC = 10240
B_SC = B - B_TC
ROWS_PER_BLOCK = 2048
NCHUNK = (C + 15) // 16


def _dense_body(x_ref, lab_ref, acc_ref):
    i = pl.program_id(0)
    x = x_ref[...]
    lab = lab_ref[0, 0, :]
    m = jnp.max(x, axis=1, keepdims=True)
    lse = m + jnp.log(jnp.sum(jnp.exp(x - m), axis=1, keepdims=True))
    col = lax.broadcasted_iota(jnp.int32, (ROWS_PER_BLOCK, C), 1)
    g = jnp.sum(jnp.where(col == lab[:, None], x, 0.0))
    partial = jnp.reshape(
        OFF * jnp.sum(x) - KLSE * jnp.sum(lse) + (CONF - OFF) * g, (1, 1)
    )

    @pl.when(i == 0)
    def _():
        acc_ref[...] = jnp.zeros((1, 1), jnp.float32)

    acc_ref[...] += partial


_dense_call = pl.pallas_call(
    _dense_body,
    grid=(B_TC // ROWS_PER_BLOCK,),
    in_specs=[
        pl.BlockSpec((ROWS_PER_BLOCK, C), lambda i: (i, 0)),
        pl.BlockSpec((1, 1, ROWS_PER_BLOCK), lambda i: (i, 0, 0)),
    ],
    out_specs=pl.BlockSpec((1, 1), lambda i: (0, 0)),
    out_shape=jax.ShapeDtypeStruct((1, 1), jnp.float32),
)


_INFO = plsc.get_sparse_core_info()
_NC = _INFO.num_cores
_NS = _INFO.num_subcores
_NW = _NC * _NS
_TR = B_SC // _NW
_SLAB = 8
_NSLAB = _TR // _SLAB
_NFULL = C // 16
_TAILOFF = C - 16

_sc_mesh = plsc.VectorSubcoreMesh(core_axis_name="c", subcore_axis_name="s")


@functools.partial(
    pl.kernel,
    out_type=jax.ShapeDtypeStruct((4, B_SC, 16), jnp.float32),
    mesh=_sc_mesh,
    compiler_params=pltpu.CompilerParams(use_tc_tiling_on_sc=True),
    scratch_types=[
        pltpu.VMEM((_TR,), jnp.int32),
        pltpu.VMEM((_TR, 16), jnp.int32),
        pltpu.VMEM((_SLAB, C), jnp.float32),
        pltpu.VMEM((4, _SLAB, 16), jnp.float32),
    ],
)
def _sc_dense(x_hbm, labels_hbm, out_hbm, lab_v, splat_v, xbuf, st_v):
    cc = lax.axis_index("c")
    ss = lax.axis_index("s")
    wid = ss * _NC + cc
    r0 = B_TC + wid * _TR
    pltpu.sync_copy(labels_hbm.at[pl.ds(r0, _TR)], lab_v)
    for j in range(_TR // 16):
        chunk = lab_v[pl.ds(j * 16, 16)]
        for r in range(16):
            splat_v[j * 16 + r, :] = jnp.broadcast_to(chunk[r], (16,))
    iota = lax.iota(jnp.int32, 16)
    fresh = iota >= (16 - (C - _NFULL * 16))

    def slab_body(sl, _):
        row = r0 + sl * _SLAB
        pltpu.sync_copy(x_hbm.at[pl.ds(row, _SLAB), :], xbuf)
        for r in range(_SLAB):
            m = jnp.full((16,), -jnp.inf, jnp.float32)
            for cn in range(_NFULL):
                m = jnp.maximum(m, xbuf[r, pl.ds(cn * 16, 16)])
            xt = xbuf[r, pl.ds(_TAILOFF, 16)]
            m = jnp.maximum(m, jnp.where(fresh, xt, -jnp.inf))
            s_l = jnp.zeros((16,), jnp.float32)
            rs_l = jnp.zeros((16,), jnp.float32)
            for cn in range(_NFULL):
                x = xbuf[r, pl.ds(cn * 16, 16)]
                s_l = s_l + jnp.exp(x - m)
                rs_l = rs_l + x
            s_l = s_l + jnp.exp(jnp.where(fresh, xt, -jnp.inf) - m)
            rs_l = rs_l + jnp.where(fresh, xt, 0.0)
            lab_s = splat_v[sl * _SLAB + r, :]
            cstartv = lax.shift_left(lax.shift_right_logical(lab_s, 4), 4)
            xg = xbuf[r, pl.ds(pl.multiple_of(cstartv[0], 16), 16)]
            g_l = jnp.where(cstartv + iota == lab_s, xg, 0.0)
            st_v[0, r, :] = m
            st_v[1, r, :] = s_l
            st_v[2, r, :] = rs_l
            st_v[3, r, :] = g_l
        pltpu.sync_copy(
            st_v, out_hbm.at[:, pl.ds(wid * _TR + sl * _SLAB, _SLAB), :]
        )
        return 0

    lax.fori_loop(0, _NSLAB, slab_body, 0)


def _finish_body(st_ref, acc_ref):
    m_l = st_ref[0]
    s_l = st_ref[1]
    rs_l = st_ref[2]
    g_l = st_ref[3]
    mm = jnp.max(m_l, axis=1, keepdims=True)
    s_tot = jnp.sum(s_l * jnp.exp(m_l - mm), axis=1, keepdims=True)
    lse = mm + jnp.log(s_tot)
    acc_ref[...] = jnp.reshape(
        OFF * jnp.sum(rs_l) - KLSE * jnp.sum(lse) + (CONF - OFF) * jnp.sum(g_l),
        (1, 1),
    )


_finish_call = pl.pallas_call(
    _finish_body,
    out_shape=jax.ShapeDtypeStruct((1, 1), jnp.float32),
)


@jax.jit
def kernel(outputs, labels):
    lab32 = labels.astype(jnp.int32)
    stats = _sc_dense(outputs, lab32)
    lab3 = jnp.reshape(lab32, (B // ROWS_PER_BLOCK, 1, ROWS_PER_BLOCK))
    acc_tc = _dense_call(outputs, lab3)[0, 0]
    acc_sc = _finish_call(stats)[0, 0]
    return ENT - (acc_tc + acc_sc) / B


# speedup vs baseline: 1.6265x; 1.6238x over previous
import jax
import jax.numpy as jnp
import numpy as np
from jax import lax
from jax.experimental import pallas as pl

B = 16384
C = 1000
SMOOTHING = 0.1
CONF = 1.0 - SMOOTHING
OFF = SMOOTHING / (C - 1)
KLSE = CONF - OFF + OFF * C
ENT = CONF * float(np.log(CONF)) + (C - 1) * OFF * float(np.log(OFF))

ROWS_PER_BLOCK = 2048


def _dense_body(x_ref, lab_ref, acc_ref):
    i = pl.program_id(0)
    x = x_ref[...]
    lab = lab_ref[0, 0, :]
    m = jnp.max(x, axis=1, keepdims=True)
    lse = m + jnp.log(jnp.sum(jnp.exp(x - m), axis=1, keepdims=True))
    col = lax.broadcasted_iota(jnp.int32, (ROWS_PER_BLOCK, C), 1)
    g = jnp.sum(jnp.where(col == lab[:, None], x, 0.0))
    partial = jnp.reshape(
        OFF * jnp.sum(x) - KLSE * jnp.sum(lse) + (CONF - OFF) * g, (1, 1)
    )

    @pl.when(i == 0)
    def _():
        acc_ref[...] = jnp.zeros((1, 1), jnp.float32)

    acc_ref[...] += partial


_dense_call = pl.pallas_call(
    _dense_body,
    grid=(B // ROWS_PER_BLOCK,),
    in_specs=[
        pl.BlockSpec((ROWS_PER_BLOCK, C), lambda i: (i, 0)),
        pl.BlockSpec((1, 1, ROWS_PER_BLOCK), lambda i: (i, 0, 0)),
    ],
    out_specs=pl.BlockSpec((1, 1), lambda i: (0, 0)),
    out_shape=jax.ShapeDtypeStruct((1, 1), jnp.float32),
)


@jax.jit
def kernel(outputs, labels):
    lab3 = jnp.reshape(labels.astype(jnp.int32), (B // ROWS_PER_BLOCK, 1, ROWS_PER_BLOCK))
    acc = _dense_call(outputs, lab3)[0, 0]
    return ENT - acc / B


# speedup vs baseline: 3.7262x; 2.2910x over previous
import jax
import jax.numpy as jnp
import numpy as np
from jax import lax
from jax.experimental import pallas as pl

B = 16384
C = 1000
SMOOTHING = 0.1
CONF = 1.0 - SMOOTHING
OFF = SMOOTHING / (C - 1)
KLSE = CONF - OFF + OFF * C
ENT = CONF * float(np.log(CONF)) + (C - 1) * OFF * float(np.log(OFF))

COLS_PER_BLOCK = 2048


def _dense_body(x_ref, lab_ref, acc_ref):
    i = pl.program_id(0)
    x = x_ref[...]
    lab = lab_ref[0, 0, :]
    m = jnp.max(x, axis=0, keepdims=True)
    lse = m + jnp.log(jnp.sum(jnp.exp(x - m), axis=0, keepdims=True))
    row = lax.broadcasted_iota(jnp.int32, (C, COLS_PER_BLOCK), 0)
    g = jnp.sum(jnp.where(row == lab[None, :], x, 0.0))
    partial = jnp.reshape(
        OFF * jnp.sum(x) - KLSE * jnp.sum(lse) + (CONF - OFF) * g, (1, 1)
    )

    @pl.when(i == 0)
    def _():
        acc_ref[...] = jnp.zeros((1, 1), jnp.float32)

    acc_ref[...] += partial


_dense_call = pl.pallas_call(
    _dense_body,
    grid=(B // COLS_PER_BLOCK,),
    in_specs=[
        pl.BlockSpec((C, COLS_PER_BLOCK), lambda i: (0, i)),
        pl.BlockSpec((1, 1, COLS_PER_BLOCK), lambda i: (i, 0, 0)),
    ],
    out_specs=pl.BlockSpec((1, 1), lambda i: (0, 0)),
    out_shape=jax.ShapeDtypeStruct((1, 1), jnp.float32),
)


@jax.jit
def kernel(outputs, labels):
    xt = jnp.transpose(outputs)
    lab3 = jnp.reshape(
        labels.astype(jnp.int32), (B // COLS_PER_BLOCK, 1, COLS_PER_BLOCK)
    )
    acc = _dense_call(xt, lab3)[0, 0]
    return ENT - acc / B
